# baseline (speedup 1.0000x reference)
import functools

import jax
import jax.numpy as jnp
from jax import lax
from jax.experimental import pallas as pl
from jax.experimental.pallas import tpu as pltpu
from jax.experimental.pallas import tpu_sc as plsc

_NC = 2
_NS = 16
_LANES = 16
_NW = _NC * _NS


def _row_sums_sc(ids_flat, table, b, s, d, row_base):
  bpw = b // _NW
  half = bpw // 2
  chunks = []
  off = 0
  while off < s:
    sz = min(128, s - off)
    chunks.append((off, sz))
    off += sz
  nacc = d // _LANES
  inv_s = 1.0 / s
  nfull = s // _LANES
  tail = s - nfull * _LANES
  spad = s + (_LANES - tail if tail else 0)

  mesh = plsc.VectorSubcoreMesh(
      core_axis_name="c", subcore_axis_name="s", num_cores=_NC,
      num_subcores=_NS)

  @functools.partial(
      pl.kernel,
      out_type=(jax.ShapeDtypeStruct((b, d), jnp.float32),
                jax.ShapeDtypeStruct((b, d), jnp.float32),
                jax.ShapeDtypeStruct((b, _LANES), jnp.float32)),
      mesh=mesh,
      scratch_types=[
          pltpu.VMEM((spad,), jnp.int32),
          pltpu.VMEM((spad,), jnp.int32),
          pltpu.VMEM((s, d), jnp.float32),
          pltpu.VMEM((s, d), jnp.float32),
          pltpu.VMEM((2, d), jnp.float32),
          pltpu.VMEM((2, d), jnp.float32),
          pltpu.VMEM((bpw, _LANES), jnp.float32),
          pltpu.SemaphoreType.DMA,
          pltpu.SemaphoreType.DMA,
          pltpu.SemaphoreType.DMA,
          pltpu.SemaphoreType.DMA,
          pltpu.SemaphoreType.DMA,
          pltpu.SemaphoreType.DMA,
      ],
  )
  def sums_kernel(ids_hbm, table_hbm, out_hbm, amp_hbm, nz_hbm, idx_a,
                  idx_b, buf0, buf1, acc, amp_acc, nz_buf, sem0, sem1,
                  sem_ia, sem_ib, sem_out, sem_amp):
    wid = lax.axis_index("s") * _NC + lax.axis_index("c")
    row0 = wid * bpw
    grow0 = row_base + row0

    lane = lax.iota(jnp.int32, _LANES)
    zvec = jnp.zeros((_LANES,), jnp.float32)
    one = jnp.ones((_LANES,), jnp.float32)

    def fire_idx(local_row, idxref, sem):
      base = pl.multiple_of((grow0 + local_row) * s, 8)
      pltpu.async_copy(ids_hbm.at[pl.ds(base, s)], idxref.at[pl.ds(0, s)],
                       sem)

    def drain_idx(idxref, sem):
      pltpu.make_async_copy(ids_hbm.at[pl.ds(0, s)], idxref.at[pl.ds(0, s)],
                            sem).wait()

    def fire_gather(idxref, bufref, sem):
      for coff, csz in chunks:
        pltpu.async_copy(
            table_hbm.at[idxref.at[pl.ds(coff, csz)]],
            bufref.at[pl.ds(coff, csz)], sem)

    def drain_gather(bufref, sem):
      for coff, csz in chunks:
        pltpu.make_async_copy(
            table_hbm.at[idx_a.at[pl.ds(coff, csz)]],
            bufref.at[pl.ds(coff, csz)], sem).wait()

    def accumulate(bufref, slot):
      def acc_body(j, accs):
        j2 = j * 2
        mid = tuple(accs[h] + bufref[j2, pl.ds(h * _LANES, _LANES)]
                    for h in range(nacc))
        return tuple(mid[h] + bufref[j2 + 1, pl.ds(h * _LANES, _LANES)]
                     for h in range(nacc))

      accs = lax.fori_loop(
          0, s // 2, acc_body,
          tuple(jnp.zeros((_LANES,), jnp.float32) for _ in range(nacc)))
      for h in range(nacc):
        acc[slot, pl.ds(h * _LANES, _LANES)] = accs[h]
        amp_acc[slot, pl.ds(h * _LANES, _LANES)] = accs[h] * inv_s

    def count_zeros(idxref):
      cnt = zvec
      for k in range(nfull):
        vals = idxref[pl.ds(k * _LANES, _LANES)]
        cnt = cnt + jnp.where(vals == 0, one, zvec)
      if tail:
        vals = idxref[pl.ds(nfull * _LANES, _LANES)]
        cnt = cnt + (jnp.where(vals == 0, one, zvec)
                     * jnp.where(lane < tail, one, zvec))
      return cnt

    fire_idx(0, idx_a, sem_ia)
    drain_idx(idx_a, sem_ia)
    fire_gather(idx_a, buf0, sem0)
    fire_idx(1, idx_b, sem_ib)

    def pair_body(g, carry):
      r0 = g * 2
      more = g < half - 1
      drain_idx(idx_b, sem_ib)
      fire_gather(idx_b, buf1, sem1)

      drain_gather(buf0, sem0)
      nz_buf[r0, pl.ds(0, _LANES)] = count_zeros(idx_a)

      @pl.when(more)
      def _():
        fire_idx(r0 + 2, idx_a, sem_ia)

      @pl.when(g > 0)
      def _():
        pltpu.make_async_copy(acc, out_hbm.at[pl.ds(0, 2)], sem_out).wait()
        pltpu.make_async_copy(amp_acc, amp_hbm.at[pl.ds(0, 2)],
                              sem_amp).wait()

      accumulate(buf0, 0)

      @pl.when(more)
      def _():
        drain_idx(idx_a, sem_ia)
        fire_gather(idx_a, buf0, sem0)

      drain_gather(buf1, sem1)
      nz_buf[r0 + 1, pl.ds(0, _LANES)] = count_zeros(idx_b)

      @pl.when(more)
      def _():
        fire_idx(r0 + 3, idx_b, sem_ib)

      accumulate(buf1, 1)
      pltpu.async_copy(acc, out_hbm.at[pl.ds(row0 + r0, 2)], sem_out)
      pltpu.async_copy(amp_acc, amp_hbm.at[pl.ds(row0 + r0, 2)], sem_amp)
      return carry

    lax.fori_loop(0, half, pair_body, 0)
    pltpu.make_async_copy(acc, out_hbm.at[pl.ds(0, 2)], sem_out).wait()
    pltpu.make_async_copy(amp_acc, amp_hbm.at[pl.ds(0, 2)], sem_amp).wait()
    pltpu.sync_copy(nz_buf, nz_hbm.at[pl.ds(row0, bpw)])

  return sums_kernel(ids_flat, table)


def _post_tc(sums, nz16, t0, b, s, h):
  bb = 512
  grid = (b // bb,)
  inv_s = 1.0 / s

  def post_kernel(sums_ref, nz_ref, t0_ref, outr_ref, outi_ref, prob_ref,
                  phase_ref):
    nz = jnp.sum(nz_ref[...], axis=1, keepdims=True)
    cnt = s - nz
    tot = sums_ref[...]
    tr = tot[:, :h]
    ti = tot[:, h:]
    t0v = t0_ref[...]
    t0r = t0v[:, :h]
    t0i = t0v[:, h:]
    good = (cnt > 0.0).astype(jnp.float32)
    scale = good / (cnt + 1e-9)
    pr = (tr - nz * t0r) * scale
    pi = (ti - nz * t0i) * scale
    s2 = jnp.sum(pr * pr + pi * pi, axis=1, keepdims=True)
    norm = jnp.sqrt(s2) + 1e-9
    outr = pr / norm
    outi = pi / norm
    outr_ref[...] = outr
    outi_ref[...] = outi
    prob_ref[...] = outr * outr + outi * outi
    phase_ref[...] = jnp.arctan2(outi, outr)

  d = 2 * h
  out_spec = pl.BlockSpec((bb, h), lambda i: (i, 0))
  return pl.pallas_call(
      post_kernel,
      grid=grid,
      in_specs=[
          pl.BlockSpec((bb, d), lambda i: (i, 0)),
          pl.BlockSpec((bb, 16), lambda i: (i, 0)),
          pl.BlockSpec((1, d), lambda i: (0, 0)),
      ],
      out_specs=[out_spec] * 4,
      out_shape=[jax.ShapeDtypeStruct((b, h), jnp.float32)] * 4,
  )(sums, nz16, t0)


def kernel(input_ids, word_table):
  b, s = input_ids.shape
  v, d = word_table.shape
  h = d // 2
  ids_flat = input_ids.reshape(-1).astype(jnp.int32)
  t0 = word_table[0:1]
  hb = b // 4
  states, probs, phases, amps = [], [], [], []
  for i in range(4):
    sums, amp_flat, nz = _row_sums_sc(ids_flat, word_table, hb, s, d, i * hb)
    outr, outi, prob, phase = _post_tc(sums, nz, t0, hb, s, h)
    states.append(jax.lax.complex(outr, outi))
    probs.append(prob)
    phases.append(phase)
    amps.append(amp_flat)
  state = jnp.concatenate(states, axis=0)
  amplitudes = jnp.concatenate(amps, axis=0).reshape(b, 2, h).transpose(
      0, 2, 1)
  return (state, amplitudes, jnp.concatenate(probs, axis=0),
          jnp.concatenate(phases, axis=0))


# speedup vs baseline: 1.0680x; 1.0680x over previous
import functools

import jax
import jax.numpy as jnp
from jax import lax
from jax.experimental import pallas as pl
from jax.experimental.pallas import tpu as pltpu
from jax.experimental.pallas import tpu_sc as plsc

_NC = 2
_NS = 16
_LANES = 16
_NW = _NC * _NS


def _row_sums_sc(ids_flat, table, b, s, d, row_base):
  bpw = b // _NW
  half = bpw // 2
  chunks = []
  off = 0
  while off < s:
    sz = min(128, s - off)
    chunks.append((off, sz))
    off += sz
  nacc = d // _LANES
  inv_s = 1.0 / s
  nfull = s // _LANES
  tail = s - nfull * _LANES
  spad = s + (_LANES - tail if tail else 0)

  mesh = plsc.VectorSubcoreMesh(
      core_axis_name="c", subcore_axis_name="s", num_cores=_NC,
      num_subcores=_NS)

  @functools.partial(
      pl.kernel,
      out_type=(jax.ShapeDtypeStruct((b, d), jnp.float32),
                jax.ShapeDtypeStruct((b, d), jnp.float32),
                jax.ShapeDtypeStruct((b, _LANES), jnp.float32)),
      mesh=mesh,
      scratch_types=[
          pltpu.VMEM((spad,), jnp.int32),
          pltpu.VMEM((spad,), jnp.int32),
          pltpu.VMEM((s, d), jnp.float32),
          pltpu.VMEM((s, d), jnp.float32),
          pltpu.VMEM((2, d), jnp.float32),
          pltpu.VMEM((2, d), jnp.float32),
          pltpu.VMEM((bpw, _LANES), jnp.float32),
          pltpu.SemaphoreType.DMA,
          pltpu.SemaphoreType.DMA,
          pltpu.SemaphoreType.DMA,
          pltpu.SemaphoreType.DMA,
          pltpu.SemaphoreType.DMA,
          pltpu.SemaphoreType.DMA,
      ],
  )
  def sums_kernel(ids_hbm, table_hbm, out_hbm, amp_hbm, nz_hbm, idx_a,
                  idx_b, buf0, buf1, acc, amp_acc, nz_buf, sem0, sem1,
                  sem_ia, sem_ib, sem_out, sem_amp):
    wid = lax.axis_index("s") * _NC + lax.axis_index("c")
    row0 = wid * bpw
    grow0 = row_base + row0

    lane = lax.iota(jnp.int32, _LANES)
    zvec = jnp.zeros((_LANES,), jnp.float32)
    one = jnp.ones((_LANES,), jnp.float32)

    def fire_idx(local_row, idxref, sem):
      base = pl.multiple_of((grow0 + local_row) * s, 8)
      pltpu.async_copy(ids_hbm.at[pl.ds(base, s)], idxref.at[pl.ds(0, s)],
                       sem)

    def drain_idx(idxref, sem):
      pltpu.make_async_copy(ids_hbm.at[pl.ds(0, s)], idxref.at[pl.ds(0, s)],
                            sem).wait()

    def fire_gather(idxref, bufref, sem):
      for coff, csz in chunks:
        pltpu.async_copy(
            table_hbm.at[idxref.at[pl.ds(coff, csz)]],
            bufref.at[pl.ds(coff, csz)], sem)

    def drain_gather(bufref, sem):
      for coff, csz in chunks:
        pltpu.make_async_copy(
            table_hbm.at[idx_a.at[pl.ds(coff, csz)]],
            bufref.at[pl.ds(coff, csz)], sem).wait()

    def accumulate(bufref, slot):
      def acc_body(j, accs):
        j2 = j * 2
        mid = tuple(accs[h] + bufref[j2, pl.ds(h * _LANES, _LANES)]
                    for h in range(nacc))
        return tuple(mid[h] + bufref[j2 + 1, pl.ds(h * _LANES, _LANES)]
                     for h in range(nacc))

      accs = lax.fori_loop(
          0, s // 2, acc_body,
          tuple(jnp.zeros((_LANES,), jnp.float32) for _ in range(nacc)))
      for h in range(nacc):
        acc[slot, pl.ds(h * _LANES, _LANES)] = accs[h]
        amp_acc[slot, pl.ds(h * _LANES, _LANES)] = accs[h] * inv_s

    def count_zeros(idxref):
      cnt = zvec
      for k in range(nfull):
        vals = idxref[pl.ds(k * _LANES, _LANES)]
        cnt = cnt + jnp.where(vals == 0, one, zvec)
      if tail:
        vals = idxref[pl.ds(nfull * _LANES, _LANES)]
        cnt = cnt + (jnp.where(vals == 0, one, zvec)
                     * jnp.where(lane < tail, one, zvec))
      return cnt

    fire_idx(0, idx_a, sem_ia)
    drain_idx(idx_a, sem_ia)
    fire_gather(idx_a, buf0, sem0)
    fire_idx(1, idx_b, sem_ib)

    def pair_body(g, carry):
      r0 = g * 2
      more = g < half - 1
      drain_idx(idx_b, sem_ib)
      fire_gather(idx_b, buf1, sem1)

      drain_gather(buf0, sem0)
      nz_buf[r0, pl.ds(0, _LANES)] = count_zeros(idx_a)

      @pl.when(more)
      def _():
        fire_idx(r0 + 2, idx_a, sem_ia)

      @pl.when(g > 0)
      def _():
        pltpu.make_async_copy(acc, out_hbm.at[pl.ds(0, 2)], sem_out).wait()
        pltpu.make_async_copy(amp_acc, amp_hbm.at[pl.ds(0, 2)],
                              sem_amp).wait()

      accumulate(buf0, 0)

      @pl.when(more)
      def _():
        drain_idx(idx_a, sem_ia)
        fire_gather(idx_a, buf0, sem0)

      drain_gather(buf1, sem1)
      nz_buf[r0 + 1, pl.ds(0, _LANES)] = count_zeros(idx_b)

      @pl.when(more)
      def _():
        fire_idx(r0 + 3, idx_b, sem_ib)

      accumulate(buf1, 1)
      pltpu.async_copy(acc, out_hbm.at[pl.ds(row0 + r0, 2)], sem_out)
      pltpu.async_copy(amp_acc, amp_hbm.at[pl.ds(row0 + r0, 2)], sem_amp)
      return carry

    lax.fori_loop(0, half, pair_body, 0)
    pltpu.make_async_copy(acc, out_hbm.at[pl.ds(0, 2)], sem_out).wait()
    pltpu.make_async_copy(amp_acc, amp_hbm.at[pl.ds(0, 2)], sem_amp).wait()
    pltpu.sync_copy(nz_buf, nz_hbm.at[pl.ds(row0, bpw)])

  return sums_kernel(ids_flat, table)


def _post_tc(sums, nz16, t0, b, s, h):
  bb = 512
  grid = (b // bb,)
  inv_s = 1.0 / s

  def post_kernel(sums_ref, nz_ref, t0_ref, outr_ref, outi_ref, prob_ref,
                  phase_ref):
    nz = jnp.sum(nz_ref[...], axis=1, keepdims=True)
    cnt = s - nz
    tot = sums_ref[...]
    tr = tot[:, :h]
    ti = tot[:, h:]
    t0v = t0_ref[...]
    t0r = t0v[:, :h]
    t0i = t0v[:, h:]
    good = (cnt > 0.0).astype(jnp.float32)
    scale = good / (cnt + 1e-9)
    pr = (tr - nz * t0r) * scale
    pi = (ti - nz * t0i) * scale
    s2 = jnp.sum(pr * pr + pi * pi, axis=1, keepdims=True)
    norm = jnp.sqrt(s2) + 1e-9
    outr = pr / norm
    outi = pi / norm
    outr_ref[...] = outr
    outi_ref[...] = outi
    prob_ref[...] = outr * outr + outi * outi
    phase_ref[...] = jnp.arctan2(outi, outr)

  d = 2 * h
  out_spec = pl.BlockSpec((bb, h), lambda i: (i, 0))
  return pl.pallas_call(
      post_kernel,
      grid=grid,
      in_specs=[
          pl.BlockSpec((bb, d), lambda i: (i, 0)),
          pl.BlockSpec((bb, 16), lambda i: (i, 0)),
          pl.BlockSpec((1, d), lambda i: (0, 0)),
      ],
      out_specs=[out_spec] * 4,
      out_shape=[jax.ShapeDtypeStruct((b, h), jnp.float32)] * 4,
  )(sums, nz16, t0)


def kernel(input_ids, word_table):
  b, s = input_ids.shape
  v, d = word_table.shape
  h = d // 2
  t0 = word_table[0:1]
  splits = (3 * b // 4, b // 4)
  states, probs, phases, amps = [], [], [], []
  base = 0
  for nb in splits:
    ids_part = input_ids[base:base + nb].reshape(-1).astype(jnp.int32)
    sums, amp_flat, nz = _row_sums_sc(ids_part, word_table, nb, s, d, 0)
    outr, outi, prob, phase = _post_tc(sums, nz, t0, nb, s, h)
    states.append(jax.lax.complex(outr, outi))
    probs.append(prob)
    phases.append(phase)
    amps.append(amp_flat)
    base += nb
  state = jnp.concatenate(states, axis=0)
  amplitudes = jnp.concatenate(amps, axis=0).reshape(b, 2, h).transpose(
      0, 2, 1)
  return (state, amplitudes, jnp.concatenate(probs, axis=0),
          jnp.concatenate(phases, axis=0))
